# Initial kernel scaffold; baseline (speedup 1.0000x reference)
import jax
import jax.numpy as jnp
from jax.experimental import pallas as pl


def kernel(x, rel_ddi, rel_dpi, edge_index_ddi, edge_index_dpi, edge_index_ppi):
    raise NotImplementedError("write your pallas kernel here")


# speedup vs baseline: 1.8866x; 1.8866x over previous
import jax
import jax.numpy as jnp
from jax import lax
from jax.experimental import pallas as pl
from jax.experimental.pallas import tpu as pltpu
from jax.experimental.pallas import tpu_sc as plsc

_D = 2048
_E = 12000
_NC = 2
_NS = 16
_NW = _NC * _NS
_EPT = _E // _NW
_B = 8
_NB = 48
_EPAD = _NB * _B
_CH = _D // 16
_LN = 16


def _edge_score_body(x_hbm, rel_hbm, idx_hbm, out_hbm,
                     idx_v, rel_v, rows_v, scores_v, sem0, sem1):
    wid = lax.axis_index("s") * _NC + lax.axis_index("c")
    pltpu.sync_copy(rel_hbm, rel_v)

    for seg in range(3):
        rrow = 0 if seg == 0 else 1
        pltpu.sync_copy(idx_hbm.at[seg, wid], idx_v)

        pltpu.async_copy(x_hbm.at[idx_v.at[0]], rows_v.at[0], sem0)
        pltpu.async_copy(x_hbm.at[idx_v.at[1]], rows_v.at[1], sem1)

        lane = jnp.arange(_LN, dtype=jnp.int32)

        @pl.loop(0, _NB, step=2)
        def _batches(b):
            svec = jnp.zeros((_LN,), jnp.float32)
            for rbuf in range(2):
                bb = b + rbuf
                sem = sem0 if rbuf == 0 else sem1
                pltpu.make_async_copy(
                    x_hbm.at[idx_v.at[bb]], rows_v.at[rbuf], sem).wait()
                buf = rows_v.at[rbuf]

                def _chunk(c, accs):
                    rl = rel_v[rrow, pl.ds(c * _LN, _LN)]
                    out = []
                    for e in range(_B):
                        h = buf[e, pl.ds(c * _LN, _LN)]
                        t = buf[e + _B, pl.ds(c * _LN, _LN)]
                        out.append(accs[e] + (h * t) * rl)
                    return tuple(out)

                accs = lax.fori_loop(
                    0, _CH, _chunk,
                    tuple(jnp.zeros((_LN,), jnp.float32) for _ in range(_B)))
                for e in range(_B):
                    s = jnp.clip(jnp.sum(accs[e], axis=0), 0.0, 1.0)
                    svec = jnp.where(lane == rbuf * _B + e, s, svec)

                @pl.when(bb + 2 < _NB)
                def _():
                    pltpu.async_copy(
                        x_hbm.at[idx_v.at[bb + 2]], rows_v.at[rbuf], sem)

            scores_v[pl.ds(b * _B, _LN)] = svec

        pltpu.sync_copy(scores_v, out_hbm.at[seg, wid])


_edge_score_sc = pl.kernel(
    _edge_score_body,
    out_type=jax.ShapeDtypeStruct((3, _NW, _EPAD), jnp.float32),
    mesh=plsc.VectorSubcoreMesh(core_axis_name="c", subcore_axis_name="s"),
    compiler_params=pltpu.CompilerParams(needs_layout_passes=False),
    scratch_types=[
        pltpu.VMEM((_NB, 2 * _B), jnp.int32),
        pltpu.VMEM((2, _D), jnp.float32),
        pltpu.VMEM((2, 2 * _B, _D), jnp.float32),
        pltpu.VMEM((_EPAD,), jnp.float32),
        pltpu.SemaphoreType.DMA,
        pltpu.SemaphoreType.DMA,
    ],
)


def _pack_indices(edge_index):
    h = jnp.pad(edge_index[0].reshape(_NW, _EPT), ((0, 0), (0, _EPAD - _EPT)))
    t = jnp.pad(edge_index[1].reshape(_NW, _EPT), ((0, 0), (0, _EPAD - _EPT)))
    return jnp.concatenate(
        [h.reshape(_NW, _NB, _B), t.reshape(_NW, _NB, _B)], axis=-1)


def kernel(x, rel_ddi, rel_dpi, edge_index_ddi, edge_index_dpi,
           edge_index_ppi):
    idx = jnp.stack([_pack_indices(edge_index_ddi),
                     _pack_indices(edge_index_dpi),
                     _pack_indices(edge_index_ppi)])
    rel = jnp.stack([rel_ddi, rel_dpi])
    out = _edge_score_sc(x, rel, idx)
    return out[:, :, :_EPT].reshape(-1)
